# baseline (speedup 1.0000x reference)
import functools

import jax
import jax.numpy as jnp
from jax import lax
from jax.experimental import pallas as pl
from jax.experimental.pallas import tpu as pltpu
from jax.experimental.pallas import tpu_sc as plsc

EMB = 64
NC = 2
NS = 16
NW = NC * NS
BT = 128
NBUF = 5
TP = BT + 1


def _gather_kernel(seq, nbt):
    n_blocks = seq * nbt
    per_w = n_blocks // NW
    mesh = plsc.VectorSubcoreMesh(
        core_axis_name="c", subcore_axis_name="s", num_cores=NC, num_subcores=NS
    )

    @functools.partial(
        pl.kernel,
        out_type=jax.ShapeDtypeStruct((seq, 8, nbt, 8, BT), jnp.float32),
        mesh=mesh,
        scratch_types=[
            pltpu.VMEM((NBUF, BT), jnp.int32),
            pltpu.VMEM((NBUF, BT, EMB), jnp.float32),
            pltpu.VMEM((NBUF, EMB, TP), jnp.float32),
            pltpu.SemaphoreType.DMA,
            pltpu.SemaphoreType.DMA,
            pltpu.SemaphoreType.DMA,
        ],
        compiler_params=pltpu.CompilerParams(
            use_tc_tiling_on_sc=False, needs_layout_passes=False
        ),
    )
    def body(ids_hbm, table_hbm, out_hbm, idx_v, g_v, t_v, isem, gsem, ssem):
        wid = lax.axis_index("s") * NC + lax.axis_index("c")
        f0 = wid * per_w
        iota16 = lax.iota(jnp.int32, 16)
        erows = [iota16 + (k * 16) for k in range(EMB // 16)]

        def transpose_block(b):
            @plsc.parallel_loop(0, BT, step=1, unroll=16)
            def token(bb):
                col = jnp.full((16,), 0, jnp.int32) + bb
                for k in range(EMB // 16):
                    v = g_v[b, bb, pl.ds(k * 16, 16)]
                    plsc.store_scatter(t_v.at[b], [erows[k], col], v)

        def group(g, _):
            for b in range(NBUF):
                f = f0 + g * NBUF + b
                l, bt = f // nbt, f % nbt

                @pl.when(g > 0)
                def _():
                    pltpu.make_async_copy(
                        t_v.at[b, :, pl.ds(0, BT)], out_hbm.at[l, 0, bt], ssem
                    ).wait()

                pltpu.async_copy(ids_hbm.at[l, bt], idx_v.at[b], isem)
            for b in range(NBUF):
                f = f0 + g * NBUF + b
                l, bt = f // nbt, f % nbt
                pltpu.make_async_copy(ids_hbm.at[l, bt], idx_v.at[b], isem).wait()
                pltpu.async_copy(table_hbm.at[idx_v.at[b]], g_v.at[b], gsem)
            for b in range(NBUF):
                f = f0 + g * NBUF + b
                l, bt = f // nbt, f % nbt
                pltpu.make_async_copy(
                    table_hbm.at[idx_v.at[b]], g_v.at[b], gsem
                ).wait()
                transpose_block(b)
                for et in range(8):
                    pltpu.async_copy(
                        t_v.at[b, pl.ds(et * 8, 8), pl.ds(0, BT)],
                        out_hbm.at[l, et, bt],
                        ssem,
                    )
            return 0

        lax.fori_loop(0, per_w // NBUF, group, 0)
        for b in range(NBUF):
            f = f0 + b
            l, bt = f // nbt, f % nbt
            pltpu.make_async_copy(
                t_v.at[b, :, pl.ds(0, BT)], out_hbm.at[l, 0, bt], ssem
            ).wait()

    return body


def kernel(input_ids, embedding_table):
    bsz, seq = input_ids.shape
    nbt = bsz // BT
    ids = (jnp.transpose(input_ids.astype(jnp.int32)) * 2).reshape(seq, nbt, BT)
    vocab = embedding_table.shape[0]
    tbl = jnp.pad(embedding_table, ((0, 0), (0, 64))).reshape(2 * vocab, EMB)
    out5 = _gather_kernel(seq, nbt)(ids, tbl)
    return out5.transpose(2, 4, 0, 1, 3).reshape(bsz, seq, EMB)


# speedup vs baseline: 1.0556x; 1.0556x over previous
import functools

import jax
import jax.numpy as jnp
from jax import lax
from jax.experimental import pallas as pl
from jax.experimental.pallas import tpu as pltpu
from jax.experimental.pallas import tpu_sc as plsc

EMB = 64
NC = 2
NS = 16
NW = NC * NS
BT = 128
NBUF = 5
TP = BT + 1


def _gather_kernel(seq, nbt):
    n_blocks = seq * nbt
    per_w = n_blocks // NW
    mesh = plsc.VectorSubcoreMesh(
        core_axis_name="c", subcore_axis_name="s", num_cores=NC, num_subcores=NS
    )

    @functools.partial(
        pl.kernel,
        out_type=jax.ShapeDtypeStruct((seq, 8, nbt, 8, BT), jnp.float32),
        mesh=mesh,
        scratch_types=[
            pltpu.VMEM((NBUF, BT), jnp.int32),
            pltpu.VMEM((NBUF, BT, EMB), jnp.float32),
            pltpu.VMEM((NBUF, EMB, TP), jnp.float32),
            pltpu.SemaphoreType.DMA,
            pltpu.SemaphoreType.DMA,
            pltpu.SemaphoreType.DMA,
        ],
        compiler_params=pltpu.CompilerParams(
            use_tc_tiling_on_sc=False, needs_layout_passes=False
        ),
    )
    def body(ids_hbm, table_hbm, out_hbm, idx_v, g_v, t_v, isem, gsem, ssem):
        wid = lax.axis_index("s") * NC + lax.axis_index("c")
        f0 = wid * per_w
        iota16 = lax.iota(jnp.int32, 16)
        erows = [iota16 + (k * 16) for k in range(EMB // 16)]

        def transpose_block(b):
            @plsc.parallel_loop(0, BT, step=1, unroll=8)
            def token(bb):
                col = jnp.full((16,), 0, jnp.int32) + bb
                for k in range(EMB // 16):
                    v = g_v[b, bb, pl.ds(k * 16, 16)]
                    plsc.store_scatter(t_v.at[b], [erows[k], col], v)

        def group(g, _):
            for b in range(NBUF):
                f = f0 + g * NBUF + b
                l, bt = f // nbt, f % nbt

                @pl.when(g > 0)
                def _():
                    pltpu.make_async_copy(
                        t_v.at[b, :, pl.ds(0, BT)], out_hbm.at[l, 0, bt], ssem
                    ).wait()

                pltpu.async_copy(ids_hbm.at[l, bt], idx_v.at[b], isem)
            for b in range(NBUF):
                f = f0 + g * NBUF + b
                l, bt = f // nbt, f % nbt
                pltpu.make_async_copy(ids_hbm.at[l, bt], idx_v.at[b], isem).wait()
                pltpu.async_copy(table_hbm.at[idx_v.at[b]], g_v.at[b], gsem)
            for b in range(NBUF):
                f = f0 + g * NBUF + b
                l, bt = f // nbt, f % nbt
                pltpu.make_async_copy(
                    table_hbm.at[idx_v.at[b]], g_v.at[b], gsem
                ).wait()
                transpose_block(b)
                for et in range(8):
                    pltpu.async_copy(
                        t_v.at[b, pl.ds(et * 8, 8), pl.ds(0, BT)],
                        out_hbm.at[l, et, bt],
                        ssem,
                    )
            return 0

        lax.fori_loop(0, per_w // NBUF, group, 0)
        for b in range(NBUF):
            f = f0 + b
            l, bt = f // nbt, f % nbt
            pltpu.make_async_copy(
                t_v.at[b, :, pl.ds(0, BT)], out_hbm.at[l, 0, bt], ssem
            ).wait()

    return body


def kernel(input_ids, embedding_table):
    bsz, seq = input_ids.shape
    nbt = bsz // BT
    ids = (jnp.transpose(input_ids.astype(jnp.int32)) * 2).reshape(seq, nbt, BT)
    vocab = embedding_table.shape[0]
    tbl = jnp.pad(embedding_table, ((0, 0), (0, 64))).reshape(2 * vocab, EMB)
    out5 = _gather_kernel(seq, nbt)(ids, tbl)
    return out5.transpose(2, 4, 0, 1, 3).reshape(bsz, seq, EMB)
